# Initial kernel scaffold; baseline (speedup 1.0000x reference)
import jax
import jax.numpy as jnp
from jax.experimental import pallas as pl


def kernel(registration_pred, registration_gt, coords):
    raise NotImplementedError("write your pallas kernel here")


# speedup vs baseline: 51.3289x; 51.3289x over previous
import functools

import jax
import jax.numpy as jnp
from jax.experimental import pallas as pl
from jax.experimental.pallas import tpu as pltpu

_N = 4096
_INF = float("inf")


def _kth_smallest(d, k, axis):
    m = jnp.min(d, axis=axis, keepdims=True)
    prev = m
    for _ in range(k - 1):
        prev = m
        m = jnp.min(jnp.where(d > m, d, _INF), axis=axis, keepdims=True)
    return m, prev


def _sqdist(rows_mat, cols_mat, query_is_col):
    cross = jnp.dot(rows_mat, cols_mat, preferred_element_type=jnp.float32)
    rowsq = (rows_mat[:, 0:1] * rows_mat[:, 0:1]
             + rows_mat[:, 1:2] * rows_mat[:, 1:2]
             + rows_mat[:, 2:3] * rows_mat[:, 2:3])
    colsq = (cols_mat[0:1, :] * cols_mat[0:1, :]
             + cols_mat[1:2, :] * cols_mat[1:2, :]
             + cols_mat[2:3, :] * cols_mat[2:3, :])
    if query_is_col:
        return (-2.0 * cross + colsq) + rowsq
    return (-2.0 * cross + rowsq) + colsq


def _curv2_body(c_ref, g_ref, ct_ref, gt_ref, out_ref):
    rows_mat = c_ref[...] + g_ref[...]
    cols_mat = ct_ref[...] + gt_ref[...]
    colp = [cols_mat[d : d + 1, :] for d in range(3)]
    rowp = [rows_mat[:, d : d + 1] for d in range(3)]
    d2 = _sqdist(rows_mat, cols_mat, query_is_col=True)
    m10, _ = _kth_smallest(d2, 10, axis=0)
    mask = d2 <= m10
    for d in range(3):
        s = jnp.sum(jnp.where(mask, rowp[d], 0.0), axis=0, keepdims=True)
        out_ref[d : d + 1, :] = (s - 10.0 * colp[d]) * jnp.float32(1.0 / 9.0)


def _main_body(c_ref, g_ref, f_ref, ct_ref, gt_ref, ft_ref, cv_ref, out_ref,
               colmin_ref, acc_ref, *, tiles):
    i = pl.program_id(0)

    @pl.when(i == 0)
    def _init():
        acc_ref[0] = 0.0
        acc_ref[1] = 0.0
        acc_ref[2] = 0.0
        colmin_ref[...] = jnp.full(colmin_ref.shape, _INF, jnp.float32)

    c_row = c_ref[...]
    f_row = f_ref[...]
    w_row = c_row + f_row
    ct_all = ct_ref[...]
    gt_all = gt_ref[...]
    ft_all = ft_ref[...]
    p2cols = ct_all + gt_all
    wcols = ct_all + ft_all

    fr = [f_row[:, d : d + 1] for d in range(3)]
    wr = [w_row[:, d : d + 1] for d in range(3)]
    fc = [ft_all[d : d + 1, :] for d in range(3)]
    wc = [wcols[d : d + 1, :] for d in range(3)]
    cv = [cv_ref[d : d + 1, :] for d in range(3)]

    d11 = _sqdist(c_row, ct_all, query_is_col=False)
    m10, m9 = _kth_smallest(d11, 10, axis=1)
    mask10 = d11 <= m10
    mask9 = d11 <= m9
    moved = []
    for d in range(3):
        s = jnp.sum(jnp.where(mask10, wc[d], 0.0), axis=1, keepdims=True)
        moved.append((s - 10.0 * wr[d]) * jnp.float32(1.0 / 9.0))
    nrm = jnp.sqrt((fc[0] - fr[0]) ** 2 + (fc[1] - fr[1]) ** 2
                   + (fc[2] - fr[2]) ** 2)
    smooth_part = jnp.sum(jnp.where(mask9, nrm, 0.0)) * jnp.float32(1.0 / 8.0)

    d12 = _sqdist(w_row, p2cols, query_is_col=False)
    d1 = jnp.min(d12, axis=1, keepdims=True)
    cham1_part = jnp.sum(d1)
    colmin_ref[...] = jnp.minimum(colmin_ref[...],
                                  jnp.min(d12, axis=0, keepdims=True))
    m5 = d1
    for _ in range(4):
        m5 = jnp.min(jnp.where(d12 > m5, d12, _INF), axis=1, keepdims=True)
    w = jnp.where(d12 <= m5, 1.0 / (d12 + 1e-8), 0.0)
    wnorm = jnp.sum(w, axis=1, keepdims=True)
    curv_part = jnp.float32(0.0)
    for d in range(3):
        inter = jnp.sum(w * cv[d], axis=1, keepdims=True) / wnorm
        curv_part = curv_part + jnp.sum((inter - moved[d]) ** 2)

    acc_ref[0] += cham1_part
    acc_ref[1] += smooth_part
    acc_ref[2] += curv_part

    @pl.when(i == tiles - 1)
    def _fin():
        cham2 = jnp.sum(colmin_ref[...])
        total = (jnp.float32(0.02) * (acc_ref[0] + cham2)
                 + jnp.float32(0.02) * acc_ref[1]
                 + jnp.float32(0.006) * acc_ref[2])
        out_ref[...] = total[None, None]


def kernel(registration_pred, registration_gt, coords):
    c = coords
    g = registration_gt[0]
    f = registration_pred[0]
    ct = c.T
    gt_ = g.T
    ft = f.T

    ta = 512
    curv2_t = pl.pallas_call(
        _curv2_body,
        grid=(_N // ta,),
        in_specs=[
            pl.BlockSpec((_N, 3), lambda i: (0, 0)),
            pl.BlockSpec((_N, 3), lambda i: (0, 0)),
            pl.BlockSpec((3, ta), lambda i: (0, i)),
            pl.BlockSpec((3, ta), lambda i: (0, i)),
        ],
        out_specs=pl.BlockSpec((3, ta), lambda i: (0, i)),
        out_shape=jax.ShapeDtypeStruct((3, _N), jnp.float32),
    )(c, g, ct, gt_)

    tb = 256
    tiles = _N // tb
    total = pl.pallas_call(
        functools.partial(_main_body, tiles=tiles),
        grid=(tiles,),
        in_specs=[
            pl.BlockSpec((tb, 3), lambda i: (i, 0)),
            pl.BlockSpec((tb, 3), lambda i: (i, 0)),
            pl.BlockSpec((tb, 3), lambda i: (i, 0)),
            pl.BlockSpec((3, _N), lambda i: (0, 0)),
            pl.BlockSpec((3, _N), lambda i: (0, 0)),
            pl.BlockSpec((3, _N), lambda i: (0, 0)),
            pl.BlockSpec((3, _N), lambda i: (0, 0)),
        ],
        out_specs=pl.BlockSpec((1, 1), lambda i: (0, 0)),
        out_shape=jax.ShapeDtypeStruct((1, 1), jnp.float32),
        scratch_shapes=[
            pltpu.VMEM((1, _N), jnp.float32),
            pltpu.SMEM((4,), jnp.float32),
        ],
    )(c, g, f, ct, gt_, ft, curv2_t)

    return total.reshape(1)
